# baseline (speedup 1.0000x reference)
import functools
import numpy as np
import jax
import jax.numpy as jnp
from jax import lax
from jax.experimental import pallas as pl
from jax.experimental.pallas import tpu as pltpu
from jax.experimental.pallas import tpu_sc as plsc

_H, _W = 128, 128
_N = _H * _W
_F = 128
_B = 16
_N_PATCHES = 8
_SEED = 42
_MIN_PATCH = (16, 16)
_MAX_PATCH = (32, 32)

_RB = 16
_S = _H // _RB


def _static_patch_coords():
    rng = np.random.default_rng(_SEED)
    coords = []
    for _ in range(_N_PATCHES):
        upper_bound = [s - p for s, p in zip((_H, _W), _MAX_PATCH)]
        lower = np.array([rng.integers(0, i) for i in upper_bound])
        ps = np.array([rng.integers(m, M) for m, M in zip(_MIN_PATCH, _MAX_PATCH)])
        upper = lower + ps
        coords.append((int(lower[0]), int(lower[1]), int(upper[0]), int(upper[1])))
    return coords


_COORDS = _static_patch_coords()
_PATCH_SIZES = [(r1 - r0) * (c1 - c0) for (r0, c0, r1, c1) in _COORDS]

_MASK_NP = np.zeros((_H, _W, 1), dtype=np.float32)
for _r0, _c0, _r1, _c1 in _COORDS:
    _MASK_NP[_r0:_r1, _c0:_c1, 0] = 1.0

_IDX_LIST = []
for (_r0, _c0, _r1, _c1) in _COORDS:
    rows = np.arange(_r0, _r1)
    cols = np.arange(_c0, _c1)
    tok = (rows[:, None] * _W + cols[None, :]).reshape(-1)
    for _b in range(_B):
        _IDX_LIST.append(_b * _N + tok)
_IDX_NP = np.concatenate(_IDX_LIST).astype(np.int32)
_N_ROWS = _IDX_NP.shape[0]

_SC_INFO = plsc.get_sparse_core_info()
_NW = _SC_INFO.num_cores * _SC_INFO.num_subcores
_CHUNK = 304
_R_PER_W = -(-_N_ROWS // (_NW * _CHUNK)) * _CHUNK
_N_PAD = _NW * _R_PER_W
_IDX_PAD = np.concatenate(
    [_IDX_NP, (np.arange(_N_PAD - _N_ROWS) * 8) % (_B * _N)]
).astype(np.int32)


def _tc_body(x_ref, mv_ref, pos_ref, m_ref, o_ref):
    s = pl.program_id(1)
    x = x_ref[0]
    pos = pos_ref[s]
    m = m_ref[s]
    repl = pos + mv_ref[0, 0][None, None, :]
    o_ref[0] = jnp.where(m > 0.0, repl, x)


def _masked_copy(X4, mv, pos4, mask):
    return pl.pallas_call(
        _tc_body,
        grid=(_B, _S),
        in_specs=[
            pl.BlockSpec((1, _RB, _W, _F), lambda b, s: (b, s, 0, 0)),
            pl.BlockSpec(memory_space=pltpu.MemorySpace.VMEM),
            pl.BlockSpec(memory_space=pltpu.MemorySpace.VMEM),
            pl.BlockSpec(memory_space=pltpu.MemorySpace.VMEM),
        ],
        out_specs=pl.BlockSpec((1, _RB, _W, _F), lambda b, s: (b, s, 0, 0)),
        out_shape=jax.ShapeDtypeStruct((_B, _H, _W, _F), jnp.float32),
    )(X4, mv, pos4, mask)


@functools.partial(
    pl.kernel,
    mesh=plsc.VectorSubcoreMesh(core_axis_name="c", subcore_axis_name="s"),
    out_type=jax.ShapeDtypeStruct((_N_PAD, _F), jnp.float32),
    scratch_types=[
        pltpu.VMEM((_CHUNK,), jnp.int32),
        pltpu.VMEM((_CHUNK, _F), jnp.float32),
        pltpu.SemaphoreType.DMA,
    ],
)
def _sc_gather(x2_hbm, idx_hbm, out_hbm, idx_v, rows_v, sem):
    nc = _SC_INFO.num_cores
    wid = lax.axis_index("s") * nc + lax.axis_index("c")
    base = wid * _R_PER_W
    for k in range(_R_PER_W // _CHUNK):
        start = base + k * _CHUNK
        pltpu.sync_copy(idx_hbm.at[pl.ds(start, _CHUNK)], idx_v)
        pltpu.async_copy(x2_hbm.at[idx_v], rows_v, sem).wait()
        pltpu.sync_copy(rows_v, out_hbm.at[pl.ds(start, _CHUNK)])


@jax.jit
def kernel(X, mask_vector, positional_embedding):
    X4 = X.reshape(_B, _H, _W, _F)
    mv = mask_vector.reshape(1, 1, _F)
    pos4 = positional_embedding.reshape(_S, _RB, _W, _F)
    mask = jnp.asarray(_MASK_NP).reshape(_S, _RB, _W, 1)

    Xm = _masked_copy(X4, mv, pos4, mask).reshape(_B, _N, _F)

    packed = _sc_gather(X.reshape(_B * _N, _F), jnp.asarray(_IDX_PAD))

    patches = []
    off = 0
    for pi in _PATCH_SIZES:
        patches.append(packed[off:off + _B * pi].reshape(_B, pi, _F))
        off += _B * pi
    return (Xm,) + tuple(patches)


# speedup vs baseline: 1.1567x; 1.1567x over previous
import functools
import numpy as np
import jax
import jax.numpy as jnp
from jax import lax
from jax.experimental import pallas as pl
from jax.experimental.pallas import tpu as pltpu
from jax.experimental.pallas import tpu_sc as plsc

_H, _W = 128, 128
_N = _H * _W
_F = 128
_B = 16
_N_PATCHES = 8
_SEED = 42
_MIN_PATCH = (16, 16)
_MAX_PATCH = (32, 32)

_RB = 16
_S = _H // _RB


def _static_patch_coords():
    rng = np.random.default_rng(_SEED)
    coords = []
    for _ in range(_N_PATCHES):
        upper_bound = [s - p for s, p in zip((_H, _W), _MAX_PATCH)]
        lower = np.array([rng.integers(0, i) for i in upper_bound])
        ps = np.array([rng.integers(m, M) for m, M in zip(_MIN_PATCH, _MAX_PATCH)])
        upper = lower + ps
        coords.append((int(lower[0]), int(lower[1]), int(upper[0]), int(upper[1])))
    return coords


_COORDS = _static_patch_coords()
_PATCH_SIZES = [(r1 - r0) * (c1 - c0) for (r0, c0, r1, c1) in _COORDS]

_MASK_NP = np.zeros((_H, _W, 1), dtype=np.float32)
for _r0, _c0, _r1, _c1 in _COORDS:
    _MASK_NP[_r0:_r1, _c0:_c1, 0] = 1.0

_IDX_LIST = []
for (_r0, _c0, _r1, _c1) in _COORDS:
    rows = np.arange(_r0, _r1)
    cols = np.arange(_c0, _c1)
    tok = (rows[:, None] * _W + cols[None, :]).reshape(-1)
    for _b in range(_B):
        _IDX_LIST.append(_b * _N + tok)
_IDX_NP = np.concatenate(_IDX_LIST).astype(np.int32)

_SC_INFO = plsc.get_sparse_core_info()
_NW = _SC_INFO.num_cores * _SC_INFO.num_subcores

_TOTS = [_B * pi for pi in _PATCH_SIZES]
_LPS = [-(-t // _NW // 8) * 8 for t in _TOTS]
_OFFS = list(np.cumsum([0] + _TOTS[:-1]))
_LMAX = max(_LPS)


def _tc_body(x_ref, mv_ref, pos_ref, m_ref, o_ref):
    s = pl.program_id(1)
    x = x_ref[0]
    pos = pos_ref[s]
    m = m_ref[s]
    repl = pos + mv_ref[0, 0][None, None, :]
    o_ref[0] = jnp.where(m > 0.0, repl, x)


def _masked_copy(X4, mv, pos4, mask):
    return pl.pallas_call(
        _tc_body,
        grid=(_B, _S),
        in_specs=[
            pl.BlockSpec((1, _RB, _W, _F), lambda b, s: (b, s, 0, 0)),
            pl.BlockSpec(memory_space=pltpu.MemorySpace.VMEM),
            pl.BlockSpec(memory_space=pltpu.MemorySpace.VMEM),
            pl.BlockSpec(memory_space=pltpu.MemorySpace.VMEM),
        ],
        out_specs=pl.BlockSpec((1, _RB, _W, _F), lambda b, s: (b, s, 0, 0)),
        out_shape=jax.ShapeDtypeStruct((_B, _H, _W, _F), jnp.float32),
    )(X4, mv, pos4, mask)


@functools.partial(
    pl.kernel,
    mesh=plsc.VectorSubcoreMesh(core_axis_name="c", subcore_axis_name="s"),
    out_type=tuple(
        jax.ShapeDtypeStruct((t, _F), jnp.float32) for t in _TOTS
    ),
    scratch_types=[
        pltpu.VMEM((_LMAX,), jnp.int32),
        pltpu.VMEM((_LMAX,), jnp.int32),
        pltpu.VMEM((_LMAX, _F), jnp.float32),
        pltpu.VMEM((_LMAX, _F), jnp.float32),
        pltpu.SemaphoreType.DMA,
        pltpu.SemaphoreType.DMA,
        pltpu.SemaphoreType.DMA,
    ],
)
def _sc_gather(x2_hbm, idx_hbm, *rest):
    outs = rest[:_N_PATCHES]
    idx_bufs = rest[_N_PATCHES:_N_PATCHES + 2]
    row_bufs = rest[_N_PATCHES + 2:_N_PATCHES + 4]
    gsem = rest[_N_PATCHES + 4]
    ssems = rest[_N_PATCHES + 5:_N_PATCHES + 7]

    nc = _SC_INFO.num_cores
    wid = lax.axis_index("s") * nc + lax.axis_index("c")
    store_handles = []
    for p in range(_N_PATCHES):
        lp, tot, off = _LPS[p], _TOTS[p], _OFFS[p]
        buf = p % 2
        if p >= 2:
            store_handles[p - 2].wait()
        start = lax.min(wid * lp, tot - lp)
        pltpu.sync_copy(
            idx_hbm.at[pl.ds(off + start, lp)],
            idx_bufs[buf].at[pl.ds(0, lp)],
        )
        pltpu.async_copy(
            x2_hbm.at[idx_bufs[buf].at[pl.ds(0, lp)]],
            row_bufs[buf].at[pl.ds(0, lp)],
            gsem,
        ).wait()
        store_handles.append(
            pltpu.async_copy(
                row_bufs[buf].at[pl.ds(0, lp)],
                outs[p].at[pl.ds(start, lp)],
                ssems[buf],
            )
        )
    store_handles[-2].wait()
    store_handles[-1].wait()


@jax.jit
def kernel(X, mask_vector, positional_embedding):
    X4 = X.reshape(_B, _H, _W, _F)
    mv = mask_vector.reshape(1, 1, _F)
    pos4 = positional_embedding.reshape(_S, _RB, _W, _F)
    mask = jnp.asarray(_MASK_NP).reshape(_S, _RB, _W, 1)

    packs = _sc_gather(X.reshape(_B * _N, _F), jnp.asarray(_IDX_NP))
    Xm = _masked_copy(X4, mv, pos4, mask).reshape(_B, _N, _F)

    patches = tuple(
        p.reshape(_B, pi, _F) for p, pi in zip(packs, _PATCH_SIZES)
    )
    return (Xm,) + patches
